# baseline (speedup 1.0000x reference)
import functools

import jax
import jax.numpy as jnp
from jax import lax
from jax.experimental import pallas as pl
from jax.experimental.pallas import tpu as pltpu
from jax.experimental.pallas import tpu_sc as plsc

D = 32
K = 8192
N = 8192
NT = 4096
KT = 2048
BETA = 0.25

_NC = 2
_NW = _NC * 16
BPW = N // _NW


def _argmin_body(lhs_ref, emb_ref, xsq_ref, esq_ref, idx_ref, dsum_ref):
    lhs = lhs_ref[...]
    x_sq = xsq_ref[...]

    best_v = jnp.full((NT, 1), jnp.inf, dtype=jnp.float32)
    best_e = jnp.full((NT, 1), jnp.inf, dtype=jnp.float32)
    best_i = jnp.zeros((NT, 1), dtype=jnp.int32)
    for w in range(K // KT):
        emb = emb_ref[pl.ds(w * KT, KT), :]
        e_sq = esq_ref[:, pl.ds(w * KT, KT)]
        mm = lax.dot_general(lhs, emb, (((1,), (1,)), ((), ())),
                             preferred_element_type=jnp.float32)
        dist = (x_sq - mm) + e_sq
        m = jnp.min(dist, axis=1, keepdims=True)
        ii = lax.broadcasted_iota(jnp.int32, dist.shape, 1)
        loc = jnp.min(jnp.where(dist == m, ii, K), axis=1, keepdims=True)
        mq = m.astype(jnp.bfloat16).astype(jnp.float32)
        upd = m < best_v
        best_e = jnp.where(upd, m, best_e)
        best_v = jnp.where(upd, mq, best_v)
        best_i = jnp.where(upd, loc + w * KT, best_i)
    idx_ref[...] = best_i

    i = pl.program_id(0)

    @pl.when(i == 0)
    def _():
        dsum_ref[...] = jnp.zeros_like(dsum_ref)

    dsum_ref[...] += jnp.sum(best_e, axis=(0, 1), keepdims=True)


def _compute_indices(lhs_bf, emb_bf, x_sq, e_sq):
    return pl.pallas_call(
        _argmin_body,
        grid=(N // NT,),
        in_specs=[
            pl.BlockSpec((NT, D), lambda i: (i, 0)),
            pl.BlockSpec((K, D), lambda i: (0, 0)),
            pl.BlockSpec((NT, 1), lambda i: (i, 0)),
            pl.BlockSpec((1, K), lambda i: (0, 0)),
        ],
        out_specs=[
            pl.BlockSpec((NT, 1), lambda i: (i, 0)),
            pl.BlockSpec((1, 1), lambda i: (0, 0)),
        ],
        out_shape=[
            jax.ShapeDtypeStruct((N, 1), jnp.int32),
            jax.ShapeDtypeStruct((1, 1), jnp.float32),
        ],
    )(lhs_bf, emb_bf, x_sq, e_sq)


_RPW = BPW // 128
_DP = 128


def _sc_body(emb_hbm, idx_hbm, zq_hbm, cnt_hbm, idx_v, rows_v, ones_v, zero_v,
             shared_cnt, sem):
    cid = lax.axis_index("c")
    sid = lax.axis_index("s")
    wid = sid * _NC + cid
    pltpu.sync_copy(idx_hbm.at[wid], idx_v)
    copies = [pltpu.async_copy(emb_hbm.at[idx_v.at[r]], rows_v.at[r], sem)
              for r in range(_RPW)]

    def ostep(i, _):
        ones_v[pl.ds(i * 16, 16)] = jnp.ones((16,), jnp.float32)
        return 0
    lax.fori_loop(0, 128 // 16, ostep, 0)

    @pl.when(sid == 0)
    def _():
        def zstep(i, _):
            zero_v[pl.ds(i * 16, 16)] = jnp.zeros((16,), jnp.float32)
            return 0
        lax.fori_loop(0, K // 16, zstep, 0)
        pltpu.sync_copy(zero_v, shared_cnt)

    plsc.subcore_barrier()
    for r in range(_RPW):
        pltpu.sync_copy(ones_v, shared_cnt.at[idx_v.at[r]], add=True)
    plsc.subcore_barrier()

    @pl.when(sid == 0)
    def _():
        pltpu.sync_copy(shared_cnt, cnt_hbm.at[cid])

    for r in range(_RPW):
        copies[r].wait()
    pltpu.sync_copy(rows_v, zq_hbm.at[pl.ds(wid * _RPW, _RPW)])


def _sc_gather_hist(emb_pad, idx_rows):
    sck = functools.partial(
        pl.kernel,
        mesh=plsc.VectorSubcoreMesh(core_axis_name="c", subcore_axis_name="s"),
        out_type=[
            jax.ShapeDtypeStruct((N // 128, 128, _DP), jnp.float32),
            jax.ShapeDtypeStruct((_NC, K), jnp.float32),
        ],
        scratch_types=[
            pltpu.VMEM((_RPW, 128), jnp.int32),
            pltpu.VMEM((_RPW, 128, _DP), jnp.float32),
            pltpu.VMEM((128,), jnp.float32),
            pltpu.VMEM((K,), jnp.float32),
            pltpu.VMEM_SHARED((K,), jnp.float32),
            pltpu.SemaphoreType.DMA,
        ],
    )(_sc_body)
    return sck(emb_pad, idx_rows)


def _final_body(cnt_ref, dsum_ref, commit_ref, perp_ref, hist_ref):
    commit_ref[...] = BETA * (dsum_ref[...] / (N * D))
    counts = jnp.sum(cnt_ref[...], axis=0, keepdims=True)
    total = jnp.maximum(jnp.sum(counts), 1.0)
    probs = counts / total
    plogp = probs * jnp.log(jnp.maximum(probs, 1e-12))
    perp_ref[...] = jnp.exp(-jnp.sum(plogp, axis=(0, 1), keepdims=True))
    hist_ref[...] = counts / total


def _final(cnt_parts, dsum):
    return pl.pallas_call(
        _final_body,
        out_shape=[
            jax.ShapeDtypeStruct((1, 1), jnp.float32),
            jax.ShapeDtypeStruct((1, 1), jnp.float32),
            jax.ShapeDtypeStruct((1, K), jnp.float32),
        ],
    )(cnt_parts, dsum)


def kernel(z_e, embedding):
    B, d, T = z_e.shape
    flat = jnp.transpose(z_e, (0, 2, 1)).reshape(B * T, d)
    x_sq = jnp.sum(flat * flat, axis=1, keepdims=True)
    e_sq = jnp.sum(embedding * embedding, axis=1)
    lhs_bf = (2.0 * flat).astype(jnp.bfloat16)
    emb_bf = embedding.astype(jnp.bfloat16)
    idx2, dsum = _compute_indices(lhs_bf, emb_bf, x_sq, e_sq[None, :])
    emb_pad = jnp.pad(embedding, ((0, 0), (0, _DP - d)))
    zq_rows, cnt_parts = _sc_gather_hist(emb_pad, idx2.reshape(_NW, _RPW, 128))
    zq_flat = zq_rows.reshape(N, _DP)[:, :D]
    commit, perp, hist = _final(cnt_parts, dsum)
    z_q_st = jnp.transpose(zq_flat.reshape(B, T, d), (0, 2, 1))
    return (z_q_st, commit.reshape(()), idx2.reshape(B, T),
            perp.reshape(()), hist.reshape(K))


# speedup vs baseline: 1.2960x; 1.2960x over previous
import functools

import jax
import jax.numpy as jnp
from jax import lax
from jax.experimental import pallas as pl
from jax.experimental.pallas import tpu as pltpu
from jax.experimental.pallas import tpu_sc as plsc

D = 32
K = 8192
N = 8192
NT = 2048
KT = 2048
BETA = 0.25

_NC = 2
_NW = _NC * 16
BPW = N // _NW


def _argmin_body(lhs_ref, emb_ref, xsq_ref, esq_ref, idx_ref, dsum_ref):
    lhs = lhs_ref[...]
    x_sq = xsq_ref[...]

    best_v = jnp.full((NT, 1), jnp.inf, dtype=jnp.float32)
    best_e = jnp.full((NT, 1), jnp.inf, dtype=jnp.float32)
    best_i = jnp.zeros((NT, 1), dtype=jnp.int32)
    for w in range(K // KT):
        emb = emb_ref[pl.ds(w * KT, KT), :]
        e_sq = esq_ref[:, pl.ds(w * KT, KT)]
        mm = lax.dot_general(lhs, emb, (((1,), (1,)), ((), ())),
                             preferred_element_type=jnp.float32)
        dist = (x_sq - mm) + e_sq
        m = jnp.min(dist, axis=1, keepdims=True)
        ii = lax.broadcasted_iota(jnp.int32, dist.shape, 1)
        loc = jnp.min(jnp.where(dist == m, ii, K), axis=1, keepdims=True)
        mq = m.astype(jnp.bfloat16).astype(jnp.float32)
        upd = m < best_v
        best_e = jnp.where(upd, m, best_e)
        best_v = jnp.where(upd, mq, best_v)
        best_i = jnp.where(upd, loc + w * KT, best_i)
    idx_ref[...] = best_i

    i = pl.program_id(0)

    @pl.when(i == 0)
    def _():
        dsum_ref[...] = jnp.zeros_like(dsum_ref)

    dsum_ref[...] += jnp.sum(best_e, axis=(0, 1), keepdims=True)


def _compute_indices(lhs_bf, emb_bf, x_sq, e_sq):
    return pl.pallas_call(
        _argmin_body,
        grid=(N // NT,),
        in_specs=[
            pl.BlockSpec((NT, D), lambda i: (i, 0)),
            pl.BlockSpec((K, D), lambda i: (0, 0)),
            pl.BlockSpec((NT, 1), lambda i: (i, 0)),
            pl.BlockSpec((1, K), lambda i: (0, 0)),
        ],
        out_specs=[
            pl.BlockSpec((NT, 1), lambda i: (i, 0)),
            pl.BlockSpec((1, 1), lambda i: (0, 0)),
        ],
        out_shape=[
            jax.ShapeDtypeStruct((N, 1), jnp.int32),
            jax.ShapeDtypeStruct((1, 1), jnp.float32),
        ],
    )(lhs_bf, emb_bf, x_sq, e_sq)


_RPW = BPW // 128
_DP = 128


def _sc_body(emb_hbm, idx_hbm, zq_hbm, cnt_hbm, idx_v, rows_v, ones_v, zero_v,
             shared_cnt, sem):
    cid = lax.axis_index("c")
    sid = lax.axis_index("s")
    wid = sid * _NC + cid
    pltpu.sync_copy(idx_hbm.at[wid], idx_v)
    copies = [pltpu.async_copy(emb_hbm.at[idx_v.at[r]], rows_v.at[r], sem)
              for r in range(_RPW)]

    def ostep(i, _):
        ones_v[pl.ds(i * 16, 16)] = jnp.ones((16,), jnp.float32)
        return 0
    lax.fori_loop(0, 128 // 16, ostep, 0)

    @pl.when(sid == 0)
    def _():
        def zstep(i, _):
            zero_v[pl.ds(i * 16, 16)] = jnp.zeros((16,), jnp.float32)
            return 0
        lax.fori_loop(0, K // 16, zstep, 0)
        pltpu.sync_copy(zero_v, shared_cnt)

    plsc.subcore_barrier()
    for r in range(_RPW):
        pltpu.sync_copy(ones_v, shared_cnt.at[idx_v.at[r]], add=True)
    plsc.subcore_barrier()

    @pl.when(sid == 0)
    def _():
        pltpu.sync_copy(shared_cnt, cnt_hbm.at[cid])

    for r in range(_RPW):
        copies[r].wait()
    pltpu.sync_copy(rows_v, zq_hbm.at[pl.ds(wid * _RPW, _RPW)])


def _sc_gather_hist(emb_pad, idx_rows):
    sck = functools.partial(
        pl.kernel,
        mesh=plsc.VectorSubcoreMesh(core_axis_name="c", subcore_axis_name="s"),
        out_type=[
            jax.ShapeDtypeStruct((N // 128, 128, _DP), jnp.float32),
            jax.ShapeDtypeStruct((_NC, K), jnp.float32),
        ],
        scratch_types=[
            pltpu.VMEM((_RPW, 128), jnp.int32),
            pltpu.VMEM((_RPW, 128, _DP), jnp.float32),
            pltpu.VMEM((128,), jnp.float32),
            pltpu.VMEM((K,), jnp.float32),
            pltpu.VMEM_SHARED((K,), jnp.float32),
            pltpu.SemaphoreType.DMA,
        ],
    )(_sc_body)
    return sck(emb_pad, idx_rows)


def _final_body(cnt_ref, dsum_ref, commit_ref, perp_ref, hist_ref):
    commit_ref[...] = BETA * (dsum_ref[...] / (N * D))
    counts = jnp.sum(cnt_ref[...], axis=0, keepdims=True)
    total = jnp.maximum(jnp.sum(counts), 1.0)
    probs = counts / total
    plogp = probs * jnp.log(jnp.maximum(probs, 1e-12))
    perp_ref[...] = jnp.exp(-jnp.sum(plogp, axis=(0, 1), keepdims=True))
    hist_ref[...] = counts / total


def _final(cnt_parts, dsum):
    return pl.pallas_call(
        _final_body,
        out_shape=[
            jax.ShapeDtypeStruct((1, 1), jnp.float32),
            jax.ShapeDtypeStruct((1, 1), jnp.float32),
            jax.ShapeDtypeStruct((1, K), jnp.float32),
        ],
    )(cnt_parts, dsum)


def kernel(z_e, embedding):
    B, d, T = z_e.shape
    flat = jnp.transpose(z_e, (0, 2, 1)).reshape(B * T, d)
    x_sq = jnp.sum(flat * flat, axis=1, keepdims=True)
    e_sq = jnp.sum(embedding * embedding, axis=1)
    lhs_bf = (2.0 * flat).astype(jnp.bfloat16)
    emb_bf = embedding.astype(jnp.bfloat16)
    idx2, dsum = _compute_indices(lhs_bf, emb_bf, x_sq, e_sq[None, :])
    emb_pad = jnp.pad(embedding, ((0, 0), (0, _DP - d)))
    zq_rows, cnt_parts = _sc_gather_hist(emb_pad, idx2.reshape(_NW, _RPW, 128))
    zq_flat = zq_rows.reshape(N, _DP)[:, :D]
    commit, perp, hist = _final(cnt_parts, dsum)
    z_q_st = jnp.transpose(zq_flat.reshape(B, T, d), (0, 2, 1))
    return (z_q_st, commit.reshape(()), idx2.reshape(B, T),
            perp.reshape(()), hist.reshape(K))
